# baseline (speedup 1.0000x reference)
import functools

import jax
import jax.numpy as jnp
from jax import lax
from jax.experimental import pallas as pl
from jax.experimental.pallas import tpu as pltpu
from jax.experimental.pallas import tpu_sc as plsc

NC = 2
NS = 16
NW = NC * NS
CH = 88
G = 8
NRING = 2

S = 1000
SAMPLE_NUM = 8
GROUP = SAMPLE_NUM + 2
TEMPERATURE = 10.0

_HIGH = lax.Precision.HIGHEST


def _segment_sum_sc(x, src_r, dst_r, zinit, n_pad, nchunk):
  d2 = x.shape[1]
  rps = n_pad // NS
  ngroup = nchunk // G
  npair = ngroup // 2
  has_peel = ngroup % 2 == 1
  fa = NRING - 1
  assert nchunk % G == 0 and npair >= 1
  assert G % NRING == 0 and G >= NRING

  def body(x_hbm, src_hbm, dst_hbm, z_hbm, out_hbm,
           src_v, dst_v, rows_v, agg_sh,
           gsem0, gsem1, gsem2, gsem3, ssem0, ssem1, dsem0, dsem1):
    c = lax.axis_index("c")
    s = lax.axis_index("s")
    gsems = (gsem0, gsem1, gsem2, gsem3)
    ssems = (ssem0, ssem1)
    dsems = (dsem0, dsem1)
    pltpu.sync_copy(z_hbm, agg_sh.at[pl.ds(s * rps, rps)])
    plsc.subcore_barrier()

    def fire_gather(b, half, jj):
      pltpu.async_copy(x_hbm.at[src_v.at[half].at[jj]], rows_v.at[b],
                       gsems[b])

    def wait_gather(b):
      pltpu.make_async_copy(x_hbm.at[pl.ds(0, CH)], rows_v.at[b],
                            gsems[b]).wait()

    def fire_idx(grp, half):
      pltpu.async_copy(src_hbm.at[c].at[s].at[pl.ds(grp * G, G)],
                       src_v.at[half], ssems[half])
      pltpu.async_copy(dst_hbm.at[c].at[s].at[pl.ds(grp * G, G)],
                       dst_v.at[half], dsems[half])

    def wait_sidx(half):
      pltpu.make_async_copy(src_hbm.at[c].at[s].at[pl.ds(0, G)],
                            src_v.at[half], ssems[half]).wait()

    def wait_didx(half):
      pltpu.make_async_copy(dst_hbm.at[c].at[s].at[pl.ds(0, G)],
                            dst_v.at[half], dsems[half]).wait()

    fire_idx(0, 0)
    wait_sidx(0)
    for jj in range(fa):
      fire_gather(jj % NRING, 0, jj)

    def group_steps(h, cross):
      wait_didx(h)
      for jj in range(G):
        b = jj % NRING
        wait_gather(b)
        tgt = jj + fa
        if tgt < G:
          fire_gather(tgt % NRING, h, tgt)
        elif cross is not False:
          def _cross(jj=jj, tgt=tgt):
            if tgt == G:
              wait_sidx(1 - h)
            fire_gather(tgt % NRING, 1 - h, tgt - G)

          if cross is None:
            _cross()
          else:
            pl.when(cross)(_cross)
        pltpu.sync_copy(rows_v.at[b], agg_sh.at[dst_v.at[h].at[jj]],
                        add=True)

    def pair(p, carry):
      g0 = 2 * p
      fire_idx(g0 + 1, 1)
      group_steps(0, None)
      nxt = p + 1 < npair
      if has_peel:
        fire_idx(g0 + 2, 0)
        group_steps(1, None)
      else:
        @pl.when(nxt)
        def _():
          fire_idx(g0 + 2, 0)

        group_steps(1, nxt)
      return carry

    lax.fori_loop(0, npair, pair, 0)
    if has_peel:
      group_steps(0, False)

    plsc.subcore_barrier()
    pltpu.sync_copy(agg_sh.at[pl.ds(s * rps, rps)],
                    out_hbm.at[c, pl.ds(s * rps, rps)])

  mesh = plsc.VectorSubcoreMesh(core_axis_name="c", subcore_axis_name="s")
  f = pl.kernel(
      body,
      out_type=jax.ShapeDtypeStruct((NC, n_pad, d2), jnp.float32),
      mesh=mesh,
      scratch_types=[
          pltpu.VMEM((2, G, CH), jnp.int32),
          pltpu.VMEM((2, G, CH), jnp.int32),
          pltpu.VMEM((NRING, CH, d2), jnp.float32),
          pltpu.VMEM_SHARED((n_pad, d2), jnp.float32),
      ] + [pltpu.SemaphoreType.DMA] * 8,
  )
  return f(x, src_r, dst_r, zinit)


def _mlp_tc(x, parts, wa, ba, wb, bb, bn):
  n, d = x.shape
  dh = wa.shape[1]
  do = wb.shape[1]
  nb = n // bn

  def body(x_ref, a0_ref, a1_ref, wa_ref, ba_ref, wb_ref, bb_ref, o_ref):
    z = x_ref[...] + a0_ref[0] + a1_ref[0]
    t = jnp.dot(z, wa_ref[...], preferred_element_type=jnp.float32,
                precision=_HIGH) + ba_ref[...]
    t = jnp.maximum(t, 0.0)
    o_ref[...] = jnp.dot(t, wb_ref[...], preferred_element_type=jnp.float32,
                         precision=_HIGH) + bb_ref[...]

  return pl.pallas_call(
      body,
      grid=(nb,),
      in_specs=[
          pl.BlockSpec((bn, d), lambda i: (i, 0)),
          pl.BlockSpec((1, bn, d), lambda i: (0, i, 0)),
          pl.BlockSpec((1, bn, d), lambda i: (1, i, 0)),
          pl.BlockSpec((d, dh), lambda i: (0, 0)),
          pl.BlockSpec((1, dh), lambda i: (0, 0)),
          pl.BlockSpec((dh, do), lambda i: (0, 0)),
          pl.BlockSpec((1, do), lambda i: (0, 0)),
      ],
      out_specs=pl.BlockSpec((bn, do), lambda i: (i, 0)),
      out_shape=jax.ShapeDtypeStruct((n, do), jnp.float32),
  )(x, parts, parts, wa, ba.reshape(1, -1), wb, bb.reshape(1, -1))


def _mlp2_pool_loss_tc(h1, parts2, wa, ba, wb, bb, idx3, bn):
  n, d = h1.shape
  dh = wa.shape[1]
  nb = n // bn
  g = S // GROUP

  def body(h1_ref, a0_ref, a1_ref, wa_ref, ba_ref, wb_ref, bb_ref, idx_ref,
           o_ref, acc_ref, cnt_ref):
    i = pl.program_id(0)

    @pl.when(i == 0)
    def _init():
      acc_ref[...] = jnp.zeros_like(acc_ref)
      cnt_ref[...] = jnp.zeros_like(cnt_ref)

    h1b = h1_ref[...]
    z = h1b + a0_ref[0] + a1_ref[0]
    t = jnp.dot(z, wa_ref[...], preferred_element_type=jnp.float32,
                precision=_HIGH) + ba_ref[...]
    t = jnp.maximum(t, 0.0)
    h2b = jnp.dot(t, wb_ref[...], preferred_element_type=jnp.float32,
                  precision=_HIGH) + bb_ref[...]
    hb = jnp.concatenate([h1b, h2b], axis=1)

    idx = idx_ref[0]
    st = lax.broadcasted_iota(jnp.int32, (S, bn), 0)
    oh = (st == idx).astype(jnp.float32)
    acc_ref[...] += jnp.dot(oh, hb, preferred_element_type=jnp.float32,
                            precision=_HIGH)
    cnt_ref[...] += jnp.sum(oh, axis=1, keepdims=True)

    @pl.when(i == nb - 1)
    def _loss():
      sub = acc_ref[...] / jnp.maximum(cnt_ref[...], 1.0)
      nrm = jnp.maximum(
          jnp.sqrt(jnp.sum(sub * sub, axis=1, keepdims=True)), 1e-8)
      subn = sub / nrm
      ii = lax.broadcasted_iota(jnp.int32, (S, S), 0)
      jj = lax.broadcasted_iota(jnp.int32, (S, S), 1)
      sel = ((ii // GROUP) * GROUP == jj).astype(jnp.float32)
      selfmat = jnp.dot(sel, subn, preferred_element_type=jnp.float32,
                        precision=_HIGH)
      sims = jnp.sum(selfmat * subn, axis=1, keepdims=True)
      off = lax.broadcasted_iota(jnp.int32, (S, 1), 0) % GROUP
      mnum = (off == 1).astype(jnp.float32)
      mden = (off >= 2).astype(jnp.float32)
      gi = lax.broadcasted_iota(jnp.int32, (g, S), 0)
      gj = lax.broadcasted_iota(jnp.int32, (g, S), 1)
      gm = (gj // GROUP == gi).astype(jnp.float32)
      num = jnp.dot(gm, sims * mnum, preferred_element_type=jnp.float32,
                    precision=_HIGH)
      den = jnp.dot(gm, jnp.exp(sims) * mden,
                    preferred_element_type=jnp.float32, precision=_HIGH)
      loss = jnp.mean(jnp.log(den) - num)
      o_ref[...] = jnp.reshape(loss, (1, 1))

  return pl.pallas_call(
      body,
      grid=(nb,),
      in_specs=[
          pl.BlockSpec((bn, d), lambda i: (i, 0)),
          pl.BlockSpec((1, bn, d), lambda i: (0, i, 0)),
          pl.BlockSpec((1, bn, d), lambda i: (1, i, 0)),
          pl.BlockSpec((d, dh), lambda i: (0, 0)),
          pl.BlockSpec((1, dh), lambda i: (0, 0)),
          pl.BlockSpec((dh, d), lambda i: (0, 0)),
          pl.BlockSpec((1, d), lambda i: (0, 0)),
          pl.BlockSpec((1, 1, bn), lambda i: (i, 0, 0)),
      ],
      out_specs=pl.BlockSpec((1, 1), lambda i: (0, 0)),
      out_shape=jax.ShapeDtypeStruct((1, 1), jnp.float32),
      scratch_shapes=[
          pltpu.VMEM((S, 2 * d), jnp.float32),
          pltpu.VMEM((S, 1), jnp.float32),
      ],
  )(h1, parts2, parts2, wa, ba.reshape(1, -1), wb, bb.reshape(1, -1), idx3)


def kernel(feature, edge_index, inputs_indices,
           W1a, b1a, W1b, b1b, W2a, b2a, W2b, b2b):
  n, d = feature.shape
  e = edge_index.shape[1]

  n_pad = ((n + 1 + NS * 8 - 1) // (NS * 8)) * (NS * 8)
  nchunk = -(-e // (NW * CH))
  nchunk = ((nchunk + G - 1) // G) * G
  e_pad = NW * nchunk * CH

  src = edge_index[0]
  dst = edge_index[1]
  pad = e_pad - e
  src_r = jnp.concatenate(
      [src, jnp.zeros((pad,), src.dtype)]).reshape(NC, NS, nchunk, CH)
  dst_r = jnp.concatenate(
      [dst, jnp.full((pad,), n, dst.dtype)]).reshape(NC, NS, nchunk, CH)
  zinit = jnp.zeros((n_pad // NS, d), jnp.float32)

  bn = 1000
  idx3 = inputs_indices.reshape(n // bn, 1, bn)

  parts1 = _segment_sum_sc(feature, src_r, dst_r, zinit, n_pad, nchunk)
  h1 = _mlp_tc(feature, parts1, W1a, b1a, W1b, b1b, bn)
  parts2 = _segment_sum_sc(h1, src_r, dst_r, zinit, n_pad, nchunk)
  loss = _mlp2_pool_loss_tc(h1, parts2, W2a, b2a, W2b, b2b, idx3, bn)
  return loss[0, 0]


# speedup vs baseline: 2.2240x; 2.2240x over previous
import functools

import jax
import jax.numpy as jnp
from jax import lax
from jax.experimental import pallas as pl
from jax.experimental.pallas import tpu as pltpu
from jax.experimental.pallas import tpu_sc as plsc

NC = 2
NS = 16
NW = NC * NS
CH = 128
G = 8
NRING = 2

S = 1000
SAMPLE_NUM = 8
GROUP = SAMPLE_NUM + 2
TEMPERATURE = 10.0

_HIGH = lax.Precision.HIGHEST
_H3 = lax.Precision.HIGH


def _segment_sum_sc(x, src_r, dst_r, zinit, n_pad, nchunk):
  d2 = x.shape[1]
  rps = n_pad // NS
  ngroup = nchunk // G
  npair = ngroup // 2
  has_peel = ngroup % 2 == 1
  fa = NRING - 1
  assert nchunk % G == 0 and npair >= 1
  assert G % NRING == 0 and G >= NRING

  def body(x_hbm, src_hbm, dst_hbm, z_hbm, out_hbm,
           src_v, dst_v, rows_v, agg_sh,
           gsem0, gsem1, gsem2, gsem3, ssem0, ssem1, dsem0, dsem1):
    c = lax.axis_index("c")
    s = lax.axis_index("s")
    gsems = (gsem0, gsem1, gsem2, gsem3)
    ssems = (ssem0, ssem1)
    dsems = (dsem0, dsem1)
    pltpu.sync_copy(z_hbm, agg_sh.at[pl.ds(s * rps, rps)])
    plsc.subcore_barrier()

    def fire_gather(b, half, jj):
      pltpu.async_copy(x_hbm.at[src_v.at[half].at[jj]], rows_v.at[b],
                       gsems[b])

    def wait_gather(b):
      pltpu.make_async_copy(x_hbm.at[pl.ds(0, CH)], rows_v.at[b],
                            gsems[b]).wait()

    def fire_idx(grp, half):
      pltpu.async_copy(src_hbm.at[c].at[s].at[pl.ds(grp * G, G)],
                       src_v.at[half], ssems[half])
      pltpu.async_copy(dst_hbm.at[c].at[s].at[pl.ds(grp * G, G)],
                       dst_v.at[half], dsems[half])

    def wait_sidx(half):
      pltpu.make_async_copy(src_hbm.at[c].at[s].at[pl.ds(0, G)],
                            src_v.at[half], ssems[half]).wait()

    def wait_didx(half):
      pltpu.make_async_copy(dst_hbm.at[c].at[s].at[pl.ds(0, G)],
                            dst_v.at[half], dsems[half]).wait()

    fire_idx(0, 0)
    wait_sidx(0)
    for jj in range(fa):
      fire_gather(jj % NRING, 0, jj)

    def group_steps(h, cross):
      wait_didx(h)
      for jj in range(G):
        b = jj % NRING
        wait_gather(b)
        tgt = jj + fa
        if tgt < G:
          fire_gather(tgt % NRING, h, tgt)
        elif cross is not False:
          def _cross(jj=jj, tgt=tgt):
            if tgt == G:
              wait_sidx(1 - h)
            fire_gather(tgt % NRING, 1 - h, tgt - G)

          if cross is None:
            _cross()
          else:
            pl.when(cross)(_cross)
        pltpu.sync_copy(rows_v.at[b], agg_sh.at[dst_v.at[h].at[jj]],
                        add=True)

    def pair(p, carry):
      g0 = 2 * p
      fire_idx(g0 + 1, 1)
      group_steps(0, None)
      nxt = p + 1 < npair
      if has_peel:
        fire_idx(g0 + 2, 0)
        group_steps(1, None)
      else:
        @pl.when(nxt)
        def _():
          fire_idx(g0 + 2, 0)

        group_steps(1, nxt)
      return carry

    lax.fori_loop(0, npair, pair, 0)
    if has_peel:
      group_steps(0, False)

    plsc.subcore_barrier()
    pltpu.sync_copy(agg_sh.at[pl.ds(s * rps, rps)],
                    out_hbm.at[c, pl.ds(s * rps, rps)])

  mesh = plsc.VectorSubcoreMesh(core_axis_name="c", subcore_axis_name="s")
  f = pl.kernel(
      body,
      out_type=jax.ShapeDtypeStruct((NC, n_pad, d2), jnp.float32),
      mesh=mesh,
      scratch_types=[
          pltpu.VMEM((2, G, CH), jnp.int32),
          pltpu.VMEM((2, G, CH), jnp.int32),
          pltpu.VMEM((NRING, CH, d2), jnp.float32),
          pltpu.VMEM_SHARED((n_pad, d2), jnp.float32),
      ] + [pltpu.SemaphoreType.DMA] * 8,
  )
  return f(x, src_r, dst_r, zinit)


def _mlp_tc(x, parts, wa, ba, wb, bb, bn):
  n, d = x.shape
  dh = wa.shape[1]
  do = wb.shape[1]
  nb = n // bn

  def body(x_ref, a0_ref, a1_ref, wa_ref, ba_ref, wb_ref, bb_ref, o_ref):
    z = x_ref[...] + a0_ref[0] + a1_ref[0]
    t = jnp.dot(z, wa_ref[...], preferred_element_type=jnp.float32,
                precision=_HIGH) + ba_ref[...]
    t = jnp.maximum(t, 0.0)
    o_ref[...] = jnp.dot(t, wb_ref[...], preferred_element_type=jnp.float32,
                         precision=_HIGH) + bb_ref[...]

  return pl.pallas_call(
      body,
      grid=(nb,),
      in_specs=[
          pl.BlockSpec((bn, d), lambda i: (i, 0)),
          pl.BlockSpec((1, bn, d), lambda i: (0, i, 0)),
          pl.BlockSpec((1, bn, d), lambda i: (1, i, 0)),
          pl.BlockSpec((d, dh), lambda i: (0, 0)),
          pl.BlockSpec((1, dh), lambda i: (0, 0)),
          pl.BlockSpec((dh, do), lambda i: (0, 0)),
          pl.BlockSpec((1, do), lambda i: (0, 0)),
      ],
      out_specs=pl.BlockSpec((bn, do), lambda i: (i, 0)),
      out_shape=jax.ShapeDtypeStruct((n, do), jnp.float32),
  )(x, parts, parts, wa, ba.reshape(1, -1), wb, bb.reshape(1, -1))


def _mlp2_pool_loss_tc(h1, parts2, wa, ba, wb, bb, idx3, bn):
  n, d = h1.shape
  dh = wa.shape[1]
  nb = n // bn
  g = S // GROUP

  def body(h1_ref, a0_ref, a1_ref, wa_ref, ba_ref, wb_ref, bb_ref, idx_ref,
           o_ref, acc_ref, cnt_ref):
    i = pl.program_id(0)

    @pl.when(i == 0)
    def _init():
      acc_ref[...] = jnp.zeros_like(acc_ref)
      cnt_ref[...] = jnp.zeros_like(cnt_ref)

    h1b = h1_ref[...]
    z = h1b + a0_ref[0] + a1_ref[0]
    t = jnp.dot(z, wa_ref[...], preferred_element_type=jnp.float32,
                precision=_HIGH) + ba_ref[...]
    t = jnp.maximum(t, 0.0)
    h2b = jnp.dot(t, wb_ref[...], preferred_element_type=jnp.float32,
                  precision=_HIGH) + bb_ref[...]
    hb = jnp.concatenate([h1b, h2b], axis=1)

    idx = idx_ref[0]
    st = lax.broadcasted_iota(jnp.int32, (S, bn), 0)
    oh = (st == idx).astype(jnp.float32)
    acc_ref[...] += jnp.dot(oh, hb, preferred_element_type=jnp.float32,
                            precision=_HIGH)
    cnt_ref[...] += jnp.sum(oh, axis=1, keepdims=True)

    @pl.when(i == nb - 1)
    def _loss():
      sub = acc_ref[...] / jnp.maximum(cnt_ref[...], 1.0)
      nrm = jnp.maximum(
          jnp.sqrt(jnp.sum(sub * sub, axis=1, keepdims=True)), 1e-8)
      subn = sub / nrm
      ii = lax.broadcasted_iota(jnp.int32, (S, S), 0)
      jj = lax.broadcasted_iota(jnp.int32, (S, S), 1)
      sel = ((ii // GROUP) * GROUP == jj).astype(jnp.float32)
      selfmat = jnp.dot(sel, subn, preferred_element_type=jnp.float32,
                        precision=_HIGH)
      sims = jnp.sum(selfmat * subn, axis=1, keepdims=True)
      off = lax.broadcasted_iota(jnp.int32, (S, 1), 0) % GROUP
      mnum = (off == 1).astype(jnp.float32)
      mden = (off >= 2).astype(jnp.float32)
      gi = lax.broadcasted_iota(jnp.int32, (g, S), 0)
      gj = lax.broadcasted_iota(jnp.int32, (g, S), 1)
      gm = (gj // GROUP == gi).astype(jnp.float32)
      num = jnp.dot(gm, sims * mnum, preferred_element_type=jnp.float32,
                    precision=_HIGH)
      den = jnp.dot(gm, jnp.exp(sims) * mden,
                    preferred_element_type=jnp.float32, precision=_HIGH)
      loss = jnp.mean(jnp.log(den) - num)
      o_ref[...] = jnp.reshape(loss, (1, 1))

  return pl.pallas_call(
      body,
      grid=(nb,),
      in_specs=[
          pl.BlockSpec((bn, d), lambda i: (i, 0)),
          pl.BlockSpec((1, bn, d), lambda i: (0, i, 0)),
          pl.BlockSpec((1, bn, d), lambda i: (1, i, 0)),
          pl.BlockSpec((d, dh), lambda i: (0, 0)),
          pl.BlockSpec((1, dh), lambda i: (0, 0)),
          pl.BlockSpec((dh, d), lambda i: (0, 0)),
          pl.BlockSpec((1, d), lambda i: (0, 0)),
          pl.BlockSpec((1, 1, bn), lambda i: (i, 0, 0)),
      ],
      out_specs=pl.BlockSpec((1, 1), lambda i: (0, 0)),
      out_shape=jax.ShapeDtypeStruct((1, 1), jnp.float32),
      scratch_shapes=[
          pltpu.VMEM((S, 2 * d), jnp.float32),
          pltpu.VMEM((S, 1), jnp.float32),
      ],
  )(h1, parts2, parts2, wa, ba.reshape(1, -1), wb, bb.reshape(1, -1), idx3)


def kernel(feature, edge_index, inputs_indices,
           W1a, b1a, W1b, b1b, W2a, b2a, W2b, b2b):
  n, d = feature.shape
  e = edge_index.shape[1]

  n_pad = ((n + 1 + NS * 8 - 1) // (NS * 8)) * (NS * 8)
  nchunk = -(-e // (NW * CH))
  nchunk = ((nchunk + G - 1) // G) * G
  e_pad = NW * nchunk * CH

  src = edge_index[0]
  dst = edge_index[1]
  pad = e_pad - e
  src_r = jnp.concatenate(
      [src, jnp.zeros((pad,), src.dtype)]).reshape(NC, NS, nchunk, CH)
  dst_r = jnp.concatenate(
      [dst, jnp.full((pad,), n, dst.dtype)]).reshape(NC, NS, nchunk, CH)
  zinit = jnp.zeros((n_pad // NS, d), jnp.float32)

  bn = 1000
  idx3 = inputs_indices.reshape(n // bn, 1, bn)

  parts1 = _segment_sum_sc(feature, src_r, dst_r, zinit, n_pad, nchunk)
  h1 = _mlp_tc(feature, parts1, W1a, b1a, W1b, b1b, bn)
  parts2 = _segment_sum_sc(h1, src_r, dst_r, zinit, n_pad, nchunk)
  loss = _mlp2_pool_loss_tc(h1, parts2, W2a, b2a, W2b, b2b, idx3, bn)
  return loss[0, 0]


# speedup vs baseline: 2.2397x; 1.0070x over previous
import functools

import jax
import jax.numpy as jnp
from jax import lax
from jax.experimental import pallas as pl
from jax.experimental.pallas import tpu as pltpu
from jax.experimental.pallas import tpu_sc as plsc

NC = 2
NS = 16
NW = NC * NS
CH = 128
G = 8
NRING = 2

S = 1000
SAMPLE_NUM = 8
GROUP = SAMPLE_NUM + 2
TEMPERATURE = 10.0

_HIGH = lax.Precision.HIGHEST
_H3 = lax.Precision.HIGH


def _segment_sum_sc(x, src_r, dst_r, zinit, n_pad, nchunk):
  d2 = x.shape[1]
  rps = n_pad // NS
  ngroup = nchunk // G
  npair = ngroup // 2
  has_peel = ngroup % 2 == 1
  fa = NRING - 1
  assert nchunk % G == 0 and npair >= 1
  assert G % NRING == 0 and G >= NRING

  def body(x_hbm, src_hbm, dst_hbm, z_hbm, out_hbm,
           src_v, dst_v, rows_v, agg_sh,
           gsem0, gsem1, gsem2, gsem3, ssem0, ssem1, dsem0, dsem1):
    c = lax.axis_index("c")
    s = lax.axis_index("s")
    gsems = (gsem0, gsem1, gsem2, gsem3)
    ssems = (ssem0, ssem1)
    dsems = (dsem0, dsem1)
    pltpu.sync_copy(z_hbm, agg_sh.at[pl.ds(s * rps, rps)])
    plsc.subcore_barrier()

    def fire_gather(b, half, jj):
      pltpu.async_copy(x_hbm.at[src_v.at[half].at[jj]], rows_v.at[b],
                       gsems[b])

    def wait_gather(b):
      pltpu.make_async_copy(x_hbm.at[pl.ds(0, CH)], rows_v.at[b],
                            gsems[b]).wait()

    def fire_idx(grp, half):
      pltpu.async_copy(src_hbm.at[c].at[s].at[pl.ds(grp * G, G)],
                       src_v.at[half], ssems[half])
      pltpu.async_copy(dst_hbm.at[c].at[s].at[pl.ds(grp * G, G)],
                       dst_v.at[half], dsems[half])

    def wait_sidx(half):
      pltpu.make_async_copy(src_hbm.at[c].at[s].at[pl.ds(0, G)],
                            src_v.at[half], ssems[half]).wait()

    def wait_didx(half):
      pltpu.make_async_copy(dst_hbm.at[c].at[s].at[pl.ds(0, G)],
                            dst_v.at[half], dsems[half]).wait()

    fire_idx(0, 0)
    wait_sidx(0)
    for jj in range(fa):
      fire_gather(jj % NRING, 0, jj)

    def group_steps(h, cross):
      wait_didx(h)
      for jj in range(G):
        b = jj % NRING
        wait_gather(b)
        tgt = jj + fa
        if tgt < G:
          fire_gather(tgt % NRING, h, tgt)
        elif cross is not False:
          def _cross(jj=jj, tgt=tgt):
            if tgt == G:
              wait_sidx(1 - h)
            fire_gather(tgt % NRING, 1 - h, tgt - G)

          if cross is None:
            _cross()
          else:
            pl.when(cross)(_cross)
        pltpu.sync_copy(rows_v.at[b], agg_sh.at[dst_v.at[h].at[jj]],
                        add=True)

    def pair(p, carry):
      g0 = 2 * p
      fire_idx(g0 + 1, 1)
      group_steps(0, None)
      nxt = p + 1 < npair
      if has_peel:
        fire_idx(g0 + 2, 0)
        group_steps(1, None)
      else:
        @pl.when(nxt)
        def _():
          fire_idx(g0 + 2, 0)

        group_steps(1, nxt)
      return carry

    lax.fori_loop(0, npair, pair, 0)
    if has_peel:
      group_steps(0, False)

    plsc.subcore_barrier()
    pltpu.sync_copy(agg_sh.at[pl.ds(s * rps, rps)],
                    out_hbm.at[c, pl.ds(s * rps, rps)])

  mesh = plsc.VectorSubcoreMesh(core_axis_name="c", subcore_axis_name="s")
  f = pl.kernel(
      body,
      out_type=jax.ShapeDtypeStruct((NC, n_pad, d2), jnp.float32),
      mesh=mesh,
      scratch_types=[
          pltpu.VMEM((2, G, CH), jnp.int32),
          pltpu.VMEM((2, G, CH), jnp.int32),
          pltpu.VMEM((NRING, CH, d2), jnp.float32),
          pltpu.VMEM_SHARED((n_pad, d2), jnp.float32),
      ] + [pltpu.SemaphoreType.DMA] * 8,
  )
  return f(x, src_r, dst_r, zinit)


def _copy_tc(x, bn):
  n, d = x.shape

  def body(x_ref, o_ref):
    o_ref[...] = x_ref[...]

  return pl.pallas_call(
      body,
      grid=(n // bn,),
      in_specs=[pl.BlockSpec((bn, d), lambda i: (i, 0))],
      out_specs=pl.BlockSpec((bn, d), lambda i: (i, 0)),
      out_shape=jax.ShapeDtypeStruct((n, d), jnp.float32),
  )(x)


def _mlp_tc(x, parts, wa, ba, wb, bb, bn):
  n, d = x.shape
  dh = wa.shape[1]
  do = wb.shape[1]
  nb = n // bn

  def body(x_ref, a0_ref, a1_ref, wa_ref, ba_ref, wb_ref, bb_ref, o_ref):
    z = x_ref[...] + a0_ref[0] + a1_ref[0]
    t = jnp.dot(z, wa_ref[...], preferred_element_type=jnp.float32,
                precision=_HIGH) + ba_ref[...]
    t = jnp.maximum(t, 0.0)
    o_ref[...] = jnp.dot(t, wb_ref[...], preferred_element_type=jnp.float32,
                         precision=_HIGH) + bb_ref[...]

  return pl.pallas_call(
      body,
      grid=(nb,),
      in_specs=[
          pl.BlockSpec((bn, d), lambda i: (i, 0)),
          pl.BlockSpec((1, bn, d), lambda i: (0, i, 0)),
          pl.BlockSpec((1, bn, d), lambda i: (1, i, 0)),
          pl.BlockSpec((d, dh), lambda i: (0, 0)),
          pl.BlockSpec((1, dh), lambda i: (0, 0)),
          pl.BlockSpec((dh, do), lambda i: (0, 0)),
          pl.BlockSpec((1, do), lambda i: (0, 0)),
      ],
      out_specs=pl.BlockSpec((bn, do), lambda i: (i, 0)),
      out_shape=jax.ShapeDtypeStruct((n, do), jnp.float32),
  )(x, parts, parts, wa, ba.reshape(1, -1), wb, bb.reshape(1, -1))


def _mlp2_pool_loss_tc(h1, parts2, wa, ba, wb, bb, idx3, bn):
  n, d = h1.shape
  dh = wa.shape[1]
  nb = n // bn
  g = S // GROUP

  def body(h1_ref, a0_ref, a1_ref, wa_ref, ba_ref, wb_ref, bb_ref, idx_ref,
           o_ref, acc_ref, cnt_ref):
    i = pl.program_id(0)

    @pl.when(i == 0)
    def _init():
      acc_ref[...] = jnp.zeros_like(acc_ref)
      cnt_ref[...] = jnp.zeros_like(cnt_ref)

    h1b = h1_ref[...]
    z = h1b + a0_ref[0] + a1_ref[0]
    t = jnp.dot(z, wa_ref[...], preferred_element_type=jnp.float32,
                precision=_HIGH) + ba_ref[...]
    t = jnp.maximum(t, 0.0)
    h2b = jnp.dot(t, wb_ref[...], preferred_element_type=jnp.float32,
                  precision=_HIGH) + bb_ref[...]
    hb = jnp.concatenate([h1b, h2b], axis=1)

    idx = idx_ref[0]
    st = lax.broadcasted_iota(jnp.int32, (S, bn), 0)
    oh = (st == idx).astype(jnp.float32)
    acc_ref[...] += jnp.dot(oh, hb, preferred_element_type=jnp.float32,
                            precision=_HIGH)
    cnt_ref[...] += jnp.sum(oh, axis=1, keepdims=True)

    @pl.when(i == nb - 1)
    def _loss():
      sub = acc_ref[...] / jnp.maximum(cnt_ref[...], 1.0)
      nrm = jnp.maximum(
          jnp.sqrt(jnp.sum(sub * sub, axis=1, keepdims=True)), 1e-8)
      subn = sub / nrm
      ii = lax.broadcasted_iota(jnp.int32, (S, S), 0)
      jj = lax.broadcasted_iota(jnp.int32, (S, S), 1)
      sel = ((ii // GROUP) * GROUP == jj).astype(jnp.float32)
      selfmat = jnp.dot(sel, subn, preferred_element_type=jnp.float32,
                        precision=_HIGH)
      sims = jnp.sum(selfmat * subn, axis=1, keepdims=True)
      off = lax.broadcasted_iota(jnp.int32, (S, 1), 0) % GROUP
      mnum = (off == 1).astype(jnp.float32)
      mden = (off >= 2).astype(jnp.float32)
      gi = lax.broadcasted_iota(jnp.int32, (g, S), 0)
      gj = lax.broadcasted_iota(jnp.int32, (g, S), 1)
      gm = (gj // GROUP == gi).astype(jnp.float32)
      num = jnp.dot(gm, sims * mnum, preferred_element_type=jnp.float32,
                    precision=_HIGH)
      den = jnp.dot(gm, jnp.exp(sims) * mden,
                    preferred_element_type=jnp.float32, precision=_HIGH)
      loss = jnp.mean(jnp.log(den) - num)
      o_ref[...] = jnp.reshape(loss, (1, 1))

  return pl.pallas_call(
      body,
      grid=(nb,),
      in_specs=[
          pl.BlockSpec((bn, d), lambda i: (i, 0)),
          pl.BlockSpec((1, bn, d), lambda i: (0, i, 0)),
          pl.BlockSpec((1, bn, d), lambda i: (1, i, 0)),
          pl.BlockSpec((d, dh), lambda i: (0, 0)),
          pl.BlockSpec((1, dh), lambda i: (0, 0)),
          pl.BlockSpec((dh, d), lambda i: (0, 0)),
          pl.BlockSpec((1, d), lambda i: (0, 0)),
          pl.BlockSpec((1, 1, bn), lambda i: (i, 0, 0)),
      ],
      out_specs=pl.BlockSpec((1, 1), lambda i: (0, 0)),
      out_shape=jax.ShapeDtypeStruct((1, 1), jnp.float32),
      scratch_shapes=[
          pltpu.VMEM((S, 2 * d), jnp.float32),
          pltpu.VMEM((S, 1), jnp.float32),
      ],
  )(h1, parts2, parts2, wa, ba.reshape(1, -1), wb, bb.reshape(1, -1), idx3)


def kernel(feature, edge_index, inputs_indices,
           W1a, b1a, W1b, b1b, W2a, b2a, W2b, b2b):
  n, d = feature.shape
  e = edge_index.shape[1]

  n_pad = ((n + 1 + NS * 8 - 1) // (NS * 8)) * (NS * 8)
  nchunk = -(-e // (NW * CH))
  nchunk = ((nchunk + G - 1) // G) * G
  e_pad = NW * nchunk * CH

  src = edge_index[0]
  dst = edge_index[1]
  pad = e_pad - e
  src_r = jnp.concatenate(
      [src, jnp.zeros((pad,), src.dtype)]).reshape(NC, NS, nchunk, CH)
  dst_r = jnp.concatenate(
      [dst, jnp.full((pad,), n, dst.dtype)]).reshape(NC, NS, nchunk, CH)
  zinit = jnp.zeros((n_pad // NS, d), jnp.float32)

  bn = 1000
  idx3 = inputs_indices.reshape(n // bn, 1, bn)

  feat = _copy_tc(feature, 2000)
  parts1 = _segment_sum_sc(feat, src_r, dst_r, zinit, n_pad, nchunk)
  h1 = _mlp_tc(feat, parts1, W1a, b1a, W1b, b1b, 2000)
  parts2 = _segment_sum_sc(h1, src_r, dst_r, zinit, n_pad, nchunk)
  loss = _mlp2_pool_loss_tc(h1, parts2, W2a, b2a, W2b, b2b, idx3, bn)
  return loss[0, 0]
